# baseline (speedup 1.0000x reference)
import functools

import jax
import jax.numpy as jnp
from jax import lax
from jax.experimental import pallas as pl
from jax.experimental.pallas import tpu as pltpu
from jax.experimental.pallas import tpu_sc as plsc

EMB = 16
BATCH = 16384
NROWS = 1000000

_info = plsc.get_sparse_core_info()
_NC, _NS = _info.num_cores, _info.num_subcores
_NW = _NC * _NS
_BPW = BATCH // _NW
_CHUNK = 128
_NCHUNK = _BPW // _CHUNK
_L = _info.num_lanes

_mesh = plsc.VectorSubcoreMesh(core_axis_name="c", subcore_axis_name="s")


@functools.partial(
    pl.kernel,
    out_type=jax.ShapeDtypeStruct((BATCH, EMB), jnp.float32),
    mesh=_mesh,
    scratch_types=[
        pltpu.VMEM((_BPW,), jnp.int32),
        pltpu.VMEM((_BPW, EMB), jnp.float32),
        pltpu.SemaphoreType.DMA,
    ],
)
def _gather_sc(id_hbm, tab_hbm, out_hbm, idx_v, rows_v, sem):
    wid = lax.axis_index("s") * _NC + lax.axis_index("c")
    base = wid * _BPW
    pltpu.sync_copy(id_hbm.at[pl.ds(base, _BPW)], idx_v)

    def fetch(g, carry):
        vec = idx_v[pl.ds(g * _L, _L)]
        copies = []
        for l in range(_L):
            row = vec[l]
            copies.append(
                pltpu.async_copy(tab_hbm.at[pl.ds(row, 1)],
                                 rows_v.at[pl.ds(g * _L + l, 1)], sem))
        for c in copies:
            c.wait()
        return carry

    lax.fori_loop(0, _BPW // _L, fetch, 0)
    pltpu.sync_copy(rows_v, out_hbm.at[pl.ds(base, _BPW)])


def _mlp_body(u_ref, i_ref, w1u_ref, w1i_ref, b1_ref, w2_ref, b2_ref,
              w3_ref, b3_ref, o_ref):
    h = jnp.dot(u_ref[...], w1u_ref[...], preferred_element_type=jnp.float32)
    h = h + jnp.dot(i_ref[...], w1i_ref[...],
                    preferred_element_type=jnp.float32)
    h = jnp.maximum(h + b1_ref[...], 0.0)
    h = jnp.dot(h, w2_ref[...], preferred_element_type=jnp.float32)
    h = jnp.maximum(h + b2_ref[...], 0.0)
    o = jnp.dot(h, w3_ref[...], preferred_element_type=jnp.float32)
    o_ref[...] = jax.nn.sigmoid(o + b3_ref[...])


def _mlp_tc(uemb, iemb, w1u, w1i, b1, w2, b2, w3, b3):
    bm = 2048
    grid = (BATCH // bm,)
    full = lambda s: pl.BlockSpec(s, lambda i: (0, 0))
    return pl.pallas_call(
        _mlp_body,
        grid=grid,
        in_specs=[
            pl.BlockSpec((bm, EMB), lambda i: (i, 0)),
            pl.BlockSpec((bm, EMB), lambda i: (i, 0)),
            full((EMB, 64)), full((EMB, 64)), full((1, 64)),
            full((64, 32)), full((1, 32)),
            full((32, 1)), full((1, 1)),
        ],
        out_specs=pl.BlockSpec((bm, 1), lambda i: (i, 0)),
        out_shape=jax.ShapeDtypeStruct((BATCH, 1), jnp.float32),
    )(uemb, iemb, w1u, w1i, b1, w2, b2, w3, b3)


def kernel(user_ids, item_ids, user_table, item_table, W1, b1, W2, b2, W3, b3):
    uid = user_ids.astype(jnp.int32)
    iid = item_ids.astype(jnp.int32)
    utabT, itabT = lax.optimization_barrier((user_table.T, item_table.T))
    utab = lax.transpose(utabT, (1, 0))
    itab = lax.transpose(itabT, (1, 0))
    uemb = _gather_sc(uid, utab)
    iemb = _gather_sc(iid, itab)
    out = _mlp_tc(uemb, iemb, W1[:EMB], W1[EMB:],
                  b1.reshape(1, 64), W2, b2.reshape(1, 32),
                  W3, b3.reshape(1, 1))
    return out.reshape(BATCH)


# speedup vs baseline: 1.1285x; 1.1285x over previous
import functools

import jax
import jax.numpy as jnp
from jax import lax
from jax.experimental import pallas as pl
from jax.experimental.pallas import tpu as pltpu
from jax.experimental.pallas import tpu_sc as plsc

EMB = 16
BATCH = 16384
NROWS = 1000000

_info = plsc.get_sparse_core_info()
_NC, _NS = _info.num_cores, _info.num_subcores
_NW = _NC * _NS
_BPW = BATCH // _NW
_CHUNK = 128
_NCHUNK = _BPW // _CHUNK
_L = _info.num_lanes

_mesh = plsc.VectorSubcoreMesh(core_axis_name="c", subcore_axis_name="s")


@functools.partial(
    pl.kernel,
    out_type=(
        jax.ShapeDtypeStruct((BATCH, EMB), jnp.float32),
        jax.ShapeDtypeStruct((BATCH, EMB), jnp.float32),
    ),
    mesh=_mesh,
    scratch_types=[
        pltpu.VMEM((_BPW,), jnp.int32),
        pltpu.VMEM((_BPW, EMB), jnp.float32),
        pltpu.SemaphoreType.DMA,
    ],
)
def _gather_sc(uid_hbm, iid_hbm, utab_hbm, itab_hbm, uout_hbm, iout_hbm,
               idx_v, rows_v, sem):
    wid = lax.axis_index("s") * _NC + lax.axis_index("c")
    base = wid * _BPW

    def one_table(id_hbm, tab_hbm, out_hbm):
        pltpu.sync_copy(id_hbm.at[pl.ds(base, _BPW)], idx_v)

        def fire(g, carry):
            vec = idx_v[pl.ds(g * _L, _L)]
            for l in range(_L):
                row = vec[l]
                pltpu.async_copy(tab_hbm.at[pl.ds(row, 1)],
                                 rows_v.at[pl.ds(g * _L + l, 1)], sem)
            return carry

        lax.fori_loop(0, _BPW // _L, fire, 0)

        def drain(r, carry):
            pltpu.make_async_copy(tab_hbm.at[pl.ds(0, 1)],
                                  rows_v.at[pl.ds(0, 1)], sem).wait()
            return carry

        lax.fori_loop(0, _BPW, drain, 0)
        pltpu.sync_copy(rows_v, out_hbm.at[pl.ds(base, _BPW)])

    one_table(uid_hbm, utab_hbm, uout_hbm)
    one_table(iid_hbm, itab_hbm, iout_hbm)


def _mlp_body(u_ref, i_ref, w1u_ref, w1i_ref, b1_ref, w2_ref, b2_ref,
              w3_ref, b3_ref, o_ref):
    h = jnp.dot(u_ref[...], w1u_ref[...], preferred_element_type=jnp.float32)
    h = h + jnp.dot(i_ref[...], w1i_ref[...],
                    preferred_element_type=jnp.float32)
    h = jnp.maximum(h + b1_ref[...], 0.0)
    h = jnp.dot(h, w2_ref[...], preferred_element_type=jnp.float32)
    h = jnp.maximum(h + b2_ref[...], 0.0)
    o = jnp.dot(h, w3_ref[...], preferred_element_type=jnp.float32)
    o_ref[...] = jax.nn.sigmoid(o + b3_ref[...])


def _mlp_tc(uemb, iemb, w1u, w1i, b1, w2, b2, w3, b3):
    bm = 2048
    grid = (BATCH // bm,)
    full = lambda s: pl.BlockSpec(s, lambda i: (0, 0))
    return pl.pallas_call(
        _mlp_body,
        grid=grid,
        in_specs=[
            pl.BlockSpec((bm, EMB), lambda i: (i, 0)),
            pl.BlockSpec((bm, EMB), lambda i: (i, 0)),
            full((EMB, 64)), full((EMB, 64)), full((1, 64)),
            full((64, 32)), full((1, 32)),
            full((32, 1)), full((1, 1)),
        ],
        out_specs=pl.BlockSpec((bm, 1), lambda i: (i, 0)),
        out_shape=jax.ShapeDtypeStruct((BATCH, 1), jnp.float32),
    )(uemb, iemb, w1u, w1i, b1, w2, b2, w3, b3)


def kernel(user_ids, item_ids, user_table, item_table, W1, b1, W2, b2, W3, b3):
    uid = user_ids.astype(jnp.int32)
    iid = item_ids.astype(jnp.int32)
    utabT, itabT = lax.optimization_barrier((user_table.T, item_table.T))
    utab = lax.transpose(utabT, (1, 0))
    itab = lax.transpose(itabT, (1, 0))
    uemb, iemb = _gather_sc(uid, iid, utab, itab)
    out = _mlp_tc(uemb, iemb, W1[:EMB], W1[EMB:],
                  b1.reshape(1, 64), W2, b2.reshape(1, 32),
                  W3, b3.reshape(1, 1))
    return out.reshape(BATCH)
